# baseline (device time: 27387 ns/iter reference)
import jax
import jax.numpy as jnp
from jax import lax
from jax.experimental import pallas as pl
from jax.experimental.pallas import tpu as pltpu

HROWS = 128


def kernel(A, B):
    m, k = A.shape
    _, n = B.shape

    def body(a_ref, b_ref, out_ref, send_ref, recv_ref, send_sems, recv_sems):
        my = lax.axis_index("i")
        nbr_a = my ^ 1
        nbr_b = 3 - my

        barrier_sem = pltpu.get_barrier_semaphore()
        for nbr in [nbr_a, nbr_b]:
            pl.semaphore_signal(
                barrier_sem, inc=1,
                device_id=(nbr,), device_id_type=pl.DeviceIdType.MESH,
            )
        pl.semaphore_wait(barrier_sem, 2)

        b_bf = b_ref[:, :].astype(jnp.bfloat16)
        for s in range(12):
            lo = (s % 4) * HROWS
            send_ref[s, :, :] = b_bf[lo:lo + HROWS, :]

        def exchange(slot, dev):
            return pltpu.make_async_remote_copy(
                src_ref=send_ref.at[slot],
                dst_ref=recv_ref.at[slot],
                send_sem=send_sems.at[slot],
                recv_sem=recv_sems.at[slot],
                device_id=(dev,),
                device_id_type=pl.DeviceIdType.MESH,
            )

        rdmas = {}
        for slot, dev in ((0, nbr_a), (2, nbr_b), (1, nbr_a), (3, nbr_b)):
            r = exchange(slot, dev)
            r.start()
            rdmas[slot] = r
        for rs, ss, dev in ((0, 4, nbr_b), (2, 6, nbr_a), (1, 5, nbr_b), (3, 7, nbr_a)):
            rdmas[rs].wait_recv()
            r = exchange(ss, dev)
            r.start()
            rdmas[ss] = r
        for rs, ss, dev in ((4, 8, nbr_a), (6, 10, nbr_b), (5, 9, nbr_a), (7, 11, nbr_b)):
            rdmas[rs].wait_recv()
            r = exchange(ss, dev)
            r.start()
            rdmas[ss] = r
        for i, rs in enumerate((8, 10, 9, 11)):
            rdmas[rs].wait_recv()
            out_ref[pl.ds(i * HROWS, HROWS), :] = (
                recv_ref[rs, :, :].astype(jnp.float32))
        out_ref[pl.ds(4 * HROWS, 4 * HROWS), :] = jnp.zeros(
            (4 * HROWS, n), jnp.float32)
        for slot in range(12):
            rdmas[slot].wait_send()

    return pl.pallas_call(
        body,
        out_shape=jax.ShapeDtypeStruct((m, n), jnp.float32),
        in_specs=[
            pl.BlockSpec(memory_space=pltpu.VMEM),
            pl.BlockSpec(memory_space=pltpu.VMEM),
        ],
        out_specs=pl.BlockSpec(memory_space=pltpu.VMEM),
        scratch_shapes=[
            pltpu.VMEM((12, HROWS, n), jnp.bfloat16),
            pltpu.VMEM((12, HROWS, n), jnp.bfloat16),
            pltpu.SemaphoreType.DMA((12,)),
            pltpu.SemaphoreType.DMA((12,)),
        ],
        compiler_params=pltpu.CompilerParams(collective_id=0),
    )(A, B)


# device time: 26992 ns/iter; 1.0146x vs baseline; 1.0146x over previous
import jax
import jax.numpy as jnp
from jax import lax
from jax.experimental import pallas as pl
from jax.experimental.pallas import tpu as pltpu

N_DEV = 4
QROWS = 256
CHUNKS = 2
CROWS = QROWS // CHUNKS
NSLOT = 6 * CHUNKS


def kernel(A, B):
    m, k = A.shape
    _, n = B.shape

    def body(a_ref, b_ref, out_ref, send_ref, recv_ref, send_sems, recv_sems):
        my = lax.axis_index("i")
        nbr_a = my ^ 1
        nbr_b = 3 - my

        f = jnp.where((my == 0) | (my == 3), 0, 1)
        g = jnp.where(my < 2, 0, 1)
        p_own = f * QROWS
        p_oth = (1 - f) * QROWS
        q_own = 2 * QROWS + g * QROWS
        q_oth = 2 * QROWS + (1 - g) * QROWS

        def slot(st, bfly, c):
            return st * 2 * CHUNKS + bfly * CHUNKS + c

        barrier_sem = pltpu.get_barrier_semaphore()
        for nbr in [nbr_a, nbr_b]:
            pl.semaphore_signal(
                barrier_sem, inc=1,
                device_id=(nbr,), device_id_type=pl.DeviceIdType.MESH,
            )

        b_bf = b_ref[:, :].astype(jnp.bfloat16)

        def cdot_bf(row_off):
            a_c = a_ref[pl.ds(row_off, CROWS), :].astype(jnp.bfloat16)
            return jnp.dot(a_c, b_bf,
                           preferred_element_type=jnp.float32
                           ).astype(jnp.bfloat16)

        def exchange(s, dev):
            return pltpu.make_async_remote_copy(
                src_ref=send_ref.at[s],
                dst_ref=recv_ref.at[s],
                send_sem=send_sems.at[s],
                recv_sem=recv_sems.at[s],
                device_id=(dev,),
                device_id_type=pl.DeviceIdType.MESH,
            )

        rdmas = {}

        def start(s, dev):
            r = exchange(s, dev)
            r.start()
            rdmas[s] = r

        send_ref[slot(0, 0, 0), :, :] = cdot_bf(p_oth)
        send_ref[slot(0, 1, 0), :, :] = cdot_bf(q_oth)
        pl.semaphore_wait(barrier_sem, 2)
        start(slot(0, 0, 0), nbr_a)
        start(slot(0, 1, 0), nbr_b)
        for c in range(1, CHUNKS):
            send_ref[slot(0, 0, c), :, :] = cdot_bf(p_oth + c * CROWS)
            start(slot(0, 0, c), nbr_a)
            send_ref[slot(0, 1, c), :, :] = cdot_bf(q_oth + c * CROWS)
            start(slot(0, 1, c), nbr_b)

        for c in range(CHUNKS):
            send_ref[slot(1, 0, c), :, :] = cdot_bf(p_own + c * CROWS)
            send_ref[slot(1, 1, c), :, :] = cdot_bf(q_own + c * CROWS)

        for c in range(CHUNKS):
            for bfly, dev in ((0, nbr_b), (1, nbr_a)):
                rs, ss = slot(0, bfly, c), slot(1, bfly, c)
                rdmas[rs].wait_recv()
                send_ref[ss, :, :] = send_ref[ss, :, :] + recv_ref[rs, :, :]
                start(ss, dev)

        for c in range(CHUNKS):
            for bfly, dev in ((0, nbr_a), (1, nbr_b)):
                rs, ss = slot(1, bfly, c), slot(2, bfly, c)
                rdmas[rs].wait_recv()
                z = send_ref[rs, :, :] + recv_ref[rs, :, :]
                send_ref[ss, :, :] = z * (1.0 / (1.0 + jnp.exp(-z)))
                start(ss, dev)

        for c in range(CHUNKS):
            out_ref[pl.ds(p_own + c * CROWS, CROWS), :] = (
                send_ref[slot(2, 0, c), :, :].astype(jnp.float32))
            out_ref[pl.ds(q_own + c * CROWS, CROWS), :] = (
                send_ref[slot(2, 1, c), :, :].astype(jnp.float32))

        for c in range(CHUNKS):
            for bfly, off in ((0, p_oth), (1, q_oth)):
                rs = slot(2, bfly, c)
                rdmas[rs].wait_recv()
                out_ref[pl.ds(off + c * CROWS, CROWS), :] = (
                    recv_ref[rs, :, :].astype(jnp.float32))

        for s in range(NSLOT):
            rdmas[s].wait_send()

    return pl.pallas_call(
        body,
        out_shape=jax.ShapeDtypeStruct((m, n), jnp.float32),
        in_specs=[
            pl.BlockSpec(memory_space=pltpu.VMEM),
            pl.BlockSpec(memory_space=pltpu.VMEM),
        ],
        out_specs=pl.BlockSpec(memory_space=pltpu.VMEM),
        scratch_shapes=[
            pltpu.VMEM((NSLOT, CROWS, n), jnp.bfloat16),
            pltpu.VMEM((NSLOT, CROWS, n), jnp.bfloat16),
            pltpu.SemaphoreType.DMA((NSLOT,)),
            pltpu.SemaphoreType.DMA((NSLOT,)),
        ],
        compiler_params=pltpu.CompilerParams(collective_id=0),
    )(A, B)
